# baseline (speedup 1.0000x reference)
import functools

import jax
import jax.numpy as jnp
from jax import lax
from jax.experimental import pallas as pl
from jax.experimental.pallas import tpu as pltpu
from jax.experimental.pallas import tpu_sc as plsc

_B = 16384
_D = 64

_info = plsc.get_sparse_core_info()
_NC = _info.num_cores
_NS = _info.num_subcores
_L = _info.num_lanes
_NW = _NC * _NS
_BPW = _B // _NW
_GROUPS = _BPW // _L
_CHUNK = 128
_NCHUNK = _BPW // _CHUNK

_mesh = plsc.VectorSubcoreMesh(core_axis_name="c", subcore_axis_name="s")


@functools.partial(
    pl.kernel,
    mesh=_mesh,
    compiler_params=pltpu.CompilerParams(needs_layout_passes=False,
                                         use_tc_tiling_on_sc=False),
    out_type=jax.ShapeDtypeStruct((_NW, _L), jnp.float32),
    scratch_types=[
        pltpu.VMEM((_NCHUNK, _CHUNK), jnp.int32),
        pltpu.VMEM((_BPW, _D), jnp.float32),
        pltpu.VMEM((_BPW, _D), jnp.float32),
        pltpu.VMEM((_L,), jnp.float32),
        pltpu.SemaphoreType.DMA,
        pltpu.SemaphoreType.DMA,
    ],
)
def _center_loss_partials(x_hbm, labels_hbm, centers_hbm, out_hbm,
                          idx_v, x_v, c_v, tot_v, gsem, xsem):
    wid = lax.axis_index("s") * _NC + lax.axis_index("c")

    xcopy = pltpu.async_copy(x_hbm.at[pl.ds(wid * _BPW, _BPW)], x_v, xsem)
    pltpu.sync_copy(labels_hbm.at[pl.ds(wid * _NCHUNK, _NCHUNK)], idx_v)
    gathers = [
        pltpu.async_copy(centers_hbm.at[idx_v.at[j]],
                         c_v.at[pl.ds(j * _CHUNK, _CHUNK)], gsem)
        for j in range(_NCHUNK)
    ]
    for g in gathers:
        g.wait()
    xcopy.wait()

    def row_body(r, tot):
        acc = jnp.zeros((_L,), jnp.float32)
        for k in range(_D // _L):
            xa = x_v[r, pl.ds(k * _L, _L)]
            ca = c_v[r, pl.ds(k * _L, _L)]
            dd = xa - ca
            acc = acc + dd * dd
        dist = jnp.sum(acc)
        dist = jnp.minimum(jnp.maximum(dist, 1e-12), 1e12)
        return tot + dist

    tot = lax.fori_loop(0, _BPW, row_body, jnp.float32(0.0))
    iota = lax.iota(jnp.int32, _L)
    tot_v[...] = jnp.where(iota < 1, tot, jnp.float32(0.0))
    pltpu.sync_copy(tot_v, out_hbm.at[wid])


def kernel(x, labels, centers):
    labels2d = labels.astype(jnp.int32).reshape(_NW * _NCHUNK, _CHUNK)
    partials = _center_loss_partials(x, labels2d, centers)
    return jnp.sum(partials) * (1.0 / _B)
